# Initial kernel scaffold; baseline (speedup 1.0000x reference)
import jax
import jax.numpy as jnp
from jax.experimental import pallas as pl


def kernel(xyz, feat, W_g, W_h, bn0_w, bn0_b):
    raise NotImplementedError("write your pallas kernel here")


# speedup vs baseline: 7.8104x; 7.8104x over previous
import functools

import jax
import jax.numpy as jnp
from jax import lax
from jax.experimental import pallas as pl
from jax.experimental.pallas import tpu as pltpu
from jax.experimental.pallas import tpu_sc as plsc

K = 32
TQ = 256


def _stats_body(feat_ref, wf_ref, bnw_ref, bnb_ref, wfs_ref, bias_ref):
    f = feat_ref[...]
    mean = jnp.mean(f, axis=(0, 2))
    var = jnp.mean(jnp.square(f - mean[None, :, None]), axis=(0, 2))
    s = bnw_ref[...].reshape(-1) / jnp.sqrt(var + 1e-5)
    bp = bnb_ref[...].reshape(-1) - mean * s
    wf = wf_ref[...]
    wfs_ref[...] = wf * s[None, :]
    bias_ref[...] = lax.dot_general(
        wf, bp[:, None], (((1,), (0,)), ((), ())),
        preferred_element_type=jnp.float32).reshape(1, -1)


def _fold_bn(feat, w_f, bn0_w, bn0_b):
    CP, CH = w_f.shape
    return pl.pallas_call(
        _stats_body,
        out_shape=(
            jax.ShapeDtypeStruct((CP, CH), jnp.float32),
            jax.ShapeDtypeStruct((1, CP), jnp.float32),
        ),
    )(feat, w_f, bn0_w.reshape(1, CH), bn0_b.reshape(1, CH))


def _feat0_knn_body(xq_ref, xa_ref, fb_ref, wfs_ref, wx_ref, bias_ref,
                    f0r_ref, idx_ref):
    n = xa_ref.shape[1]
    xq = xq_ref[0]
    xa = xa_ref[0]
    fb = fb_ref[0]
    f0r = lax.dot_general(fb, wfs_ref[...], (((0,), (1,)), ((), ())),
                          preferred_element_type=jnp.float32)
    f0r = f0r + lax.dot_general(xq, wx_ref[...], (((1,), (1,)), ((), ())),
                                preferred_element_type=jnp.float32)
    f0r_ref[0] = f0r + bias_ref[...]
    inner = lax.dot_general(xq, xa, (((1,), (1,)), ((), ())),
                            preferred_element_type=jnp.float32)
    xx = jnp.sum(xa * xa, axis=1)
    score = 2.0 * inner - xx[None, :]
    iota = lax.broadcasted_iota(jnp.int32, (TQ, n), 1)
    cols = []
    for _ in range(K):
        m = jnp.max(score, axis=1, keepdims=True)
        j = jnp.min(jnp.where(score >= m, iota, n), axis=1)
        cols.append(j)
        score = jnp.where(iota == j[:, None], -jnp.inf, score)
    idx = jnp.stack(cols, axis=1)
    idx_ref[0] = idx + pl.program_id(0) * n


def _feat0_knn(xyz, feat, wfs, wx, bias):
    B, N, _ = xyz.shape
    CH = feat.shape[1]
    CP = wfs.shape[0]
    grid = (B, N // TQ)
    return pl.pallas_call(
        _feat0_knn_body,
        grid=grid,
        in_specs=[
            pl.BlockSpec((1, TQ, 3), lambda b, i: (b, i, 0)),
            pl.BlockSpec((1, N, 3), lambda b, i: (b, 0, 0)),
            pl.BlockSpec((1, CH, TQ), lambda b, i: (b, 0, i)),
            pl.BlockSpec((CP, CH), lambda b, i: (0, 0)),
            pl.BlockSpec((CP, 3), lambda b, i: (0, 0)),
            pl.BlockSpec((1, CP), lambda b, i: (0, 0)),
        ],
        out_specs=[
            pl.BlockSpec((1, TQ, CP), lambda b, i: (b, i, 0)),
            pl.BlockSpec((1, TQ, K), lambda b, i: (b, i, 0)),
        ],
        out_shape=(
            jax.ShapeDtypeStruct((B, N, CP), jnp.float32),
            jax.ShapeDtypeStruct((B, N, K), jnp.int32),
        ),
    )(xyz, xyz, feat, wfs, wx, bias)


def _gather_max_sc(table, idx2d, n_workers, q_per_w, cp):
    rows_per_chunk = 128
    q_per_chunk = rows_per_chunk // K
    chunks = q_per_w * K // rows_per_chunk
    mesh = plsc.VectorSubcoreMesh(core_axis_name="c", subcore_axis_name="s")

    @functools.partial(
        pl.kernel,
        out_type=jax.ShapeDtypeStruct((table.shape[0], cp), jnp.float32),
        mesh=mesh,
        scratch_types=[
            pltpu.VMEM((chunks, 128), jnp.int32),
            pltpu.VMEM((rows_per_chunk, cp), jnp.float32),
            pltpu.VMEM((q_per_w, cp), jnp.float32),
            pltpu.SemaphoreType.DMA,
        ],
    )
    def kb(table_hbm, idx_hbm, out_hbm, idx_v, rows_v, out_v, sem):
        wid = lax.axis_index("s") * 2 + lax.axis_index("c")
        pltpu.sync_copy(idx_hbm.at[pl.ds(wid * chunks, chunks)], idx_v)

        def chunk_body(ch, _):
            pltpu.async_copy(table_hbm.at[idx_v.at[ch]], rows_v, sem).wait()

            def col_body(c, _):
                off = pl.ds(pl.multiple_of(c * 16, 16), 16)

                def q_body(q, _):
                    base = q * K
                    acc = rows_v[base, off]
                    for j in range(1, K):
                        acc = jnp.maximum(acc, rows_v[base + j, off])
                    out_v[ch * q_per_chunk + q, off] = acc
                    return 0

                return lax.fori_loop(0, q_per_chunk, q_body, 0)

            return lax.fori_loop(0, cp // 16, col_body, 0)

        lax.fori_loop(0, chunks, chunk_body, 0)
        pltpu.sync_copy(out_v, out_hbm.at[pl.ds(wid * q_per_w, q_per_w)])

    return kb(table, idx2d)


def _out_body(fr_ref, f0_ref, whs_ref, out_ref):
    diff = fr_ref[0] - f0_ref[0]
    out_ref[0] = lax.dot_general(whs_ref[...], diff, (((1,), (1,)), ((), ())),
                                 preferred_element_type=jnp.float32)


def _out_matmul(f_rows, f0_rows, whs):
    B, N, CP = f_rows.shape
    CH = whs.shape[0]
    return pl.pallas_call(
        _out_body,
        grid=(B, N // TQ),
        in_specs=[
            pl.BlockSpec((1, TQ, CP), lambda b, i: (b, i, 0)),
            pl.BlockSpec((1, TQ, CP), lambda b, i: (b, i, 0)),
            pl.BlockSpec((CH, CP), lambda b, i: (0, 0)),
        ],
        out_specs=pl.BlockSpec((1, CH, TQ), lambda b, i: (b, 0, i)),
        out_shape=jax.ShapeDtypeStruct((B, CH, N), jnp.float32),
    )(f_rows, f0_rows, whs)


def kernel(xyz, feat, W_g, W_h, bn0_w, bn0_b):
    B, N, _ = xyz.shape
    CH = feat.shape[1]
    CP = W_g.shape[0]
    M = W_h.shape[1] // CP

    w_f = W_g[:, :CH]
    w_x = W_g[:, CH:]
    whs = W_h.reshape(CH, M, CP).sum(axis=1)

    wfs, bias = _fold_bn(feat, w_f, bn0_w, bn0_b)
    f0_rows, idx = _feat0_knn(xyz, feat, wfs, w_x, bias)

    table = f0_rows.reshape(B * N, CP)
    idx2d = idx.reshape(B * N * K // 128, 128)
    n_workers = 32
    f_rows = _gather_max_sc(table, idx2d, n_workers, B * N // n_workers, CP)

    return _out_matmul(f_rows.reshape(B, N, CP), f0_rows, whs)


# speedup vs baseline: 10.4265x; 1.3349x over previous
import functools

import jax
import jax.numpy as jnp
from jax import lax
from jax.experimental import pallas as pl
from jax.experimental.pallas import tpu as pltpu
from jax.experimental.pallas import tpu_sc as plsc

K = 32
TQ = 256


def _stats_body(feat_ref, wf_ref, bnw_ref, bnb_ref, wfs_ref, bias_ref):
    f = feat_ref[...]
    mean = jnp.mean(f, axis=(0, 2))
    var = jnp.mean(jnp.square(f - mean[None, :, None]), axis=(0, 2))
    s = bnw_ref[...].reshape(-1) / jnp.sqrt(var + 1e-5)
    bp = bnb_ref[...].reshape(-1) - mean * s
    wf = wf_ref[...]
    wfs_ref[...] = wf * s[None, :]
    bias_ref[...] = lax.dot_general(
        wf, bp[:, None], (((1,), (0,)), ((), ())),
        preferred_element_type=jnp.float32).reshape(1, -1)


def _fold_bn(feat, w_f, bn0_w, bn0_b):
    CP, CH = w_f.shape
    return pl.pallas_call(
        _stats_body,
        out_shape=(
            jax.ShapeDtypeStruct((CP, CH), jnp.float32),
            jax.ShapeDtypeStruct((1, CP), jnp.float32),
        ),
    )(feat, w_f, bn0_w.reshape(1, CH), bn0_b.reshape(1, CH))


def _feat0_knn_body(xq_ref, xa_ref, fb_ref, wfs_ref, wx_ref, bias_ref,
                    f0r_ref, idx_ref):
    n = xa_ref.shape[1]
    xq = xq_ref[0]
    xa = xa_ref[0]
    fb = fb_ref[0]
    f0r = lax.dot_general(fb, wfs_ref[...], (((0,), (1,)), ((), ())),
                          preferred_element_type=jnp.float32)
    f0r = f0r + lax.dot_general(xq, wx_ref[...], (((1,), (1,)), ((), ())),
                                preferred_element_type=jnp.float32)
    f0r_ref[0] = f0r + bias_ref[...]
    inner = lax.dot_general(xq, xa, (((1,), (1,)), ((), ())),
                            preferred_element_type=jnp.float32)
    xx = jnp.sum(xa * xa, axis=1)
    score = 2.0 * inner - xx[None, :]
    iota_f = lax.broadcasted_iota(jnp.int32, (TQ, n), 1).astype(jnp.float32)
    cols = []
    for _ in range(K):
        m = jnp.max(score, axis=1, keepdims=True)
        ge = score >= m
        j = jnp.min(jnp.where(ge, iota_f, float(n)), axis=1)
        cols.append(j)
        score = jnp.where(ge, -jnp.inf, score)
    idx = jnp.stack(cols, axis=1).astype(jnp.int32)
    idx_ref[0] = idx + pl.program_id(0) * n


def _feat0_knn(xyz, feat, wfs, wx, bias):
    B, N, _ = xyz.shape
    CH = feat.shape[1]
    CP = wfs.shape[0]
    grid = (B, N // TQ)
    return pl.pallas_call(
        _feat0_knn_body,
        grid=grid,
        in_specs=[
            pl.BlockSpec((1, TQ, 3), lambda b, i: (b, i, 0)),
            pl.BlockSpec((1, N, 3), lambda b, i: (b, 0, 0)),
            pl.BlockSpec((1, CH, TQ), lambda b, i: (b, 0, i)),
            pl.BlockSpec((CP, CH), lambda b, i: (0, 0)),
            pl.BlockSpec((CP, 3), lambda b, i: (0, 0)),
            pl.BlockSpec((1, CP), lambda b, i: (0, 0)),
        ],
        out_specs=[
            pl.BlockSpec((1, TQ, CP), lambda b, i: (b, i, 0)),
            pl.BlockSpec((1, TQ, K), lambda b, i: (b, i, 0)),
        ],
        out_shape=(
            jax.ShapeDtypeStruct((B, N, CP), jnp.float32),
            jax.ShapeDtypeStruct((B, N, K), jnp.int32),
        ),
    )(xyz, xyz, feat, wfs, wx, bias)


def _gather_max_sc(table, idx2d, n_workers, q_per_w, cp):
    rows_per_chunk = 128
    q_per_chunk = rows_per_chunk // K
    chunks = q_per_w * K // rows_per_chunk
    mesh = plsc.VectorSubcoreMesh(core_axis_name="c", subcore_axis_name="s")

    @functools.partial(
        pl.kernel,
        out_type=jax.ShapeDtypeStruct((table.shape[0], cp), jnp.float32),
        mesh=mesh,
        scratch_types=[
            pltpu.VMEM((chunks, 128), jnp.int32),
            pltpu.VMEM((rows_per_chunk, cp), jnp.float32),
            pltpu.VMEM((rows_per_chunk, cp), jnp.float32),
            pltpu.VMEM((q_per_w, cp), jnp.float32),
            pltpu.SemaphoreType.DMA,
            pltpu.SemaphoreType.DMA,
        ],
    )
    def kb(table_hbm, idx_hbm, out_hbm, idx_v, rows0, rows1, out_v, s0, s1):
        wid = lax.axis_index("s") * 2 + lax.axis_index("c")
        pltpu.sync_copy(idx_hbm.at[pl.ds(wid * chunks, chunks)], idx_v)
        bufs = (rows0, rows1)
        sems = (s0, s1)
        pltpu.async_copy(table_hbm.at[idx_v.at[0]], rows0, s0)
        pltpu.async_copy(table_hbm.at[idx_v.at[1]], rows1, s1)

        def pair_body(p, _):
            for b in range(2):
                ch = p * 2 + b
                rows_v, sem = bufs[b], sems[b]
                pltpu.make_async_copy(table_hbm.at[idx_v.at[ch]], rows_v,
                                      sem).wait()

                def col_body(c, _):
                    off = pl.ds(pl.multiple_of(c * 16, 16), 16)
                    for q in range(q_per_chunk):
                        vals = [rows_v[q * K + j, off] for j in range(K)]
                        while len(vals) > 1:
                            vals = [jnp.maximum(vals[i], vals[i + 1])
                                    for i in range(0, len(vals) - 1, 2)] + (
                                        [vals[-1]] if len(vals) % 2 else [])
                        out_v[ch * q_per_chunk + q, off] = vals[0]
                    return 0

                lax.fori_loop(0, cp // 16, col_body, 0)

                @pl.when(ch + 2 < chunks)
                def _():
                    pltpu.async_copy(table_hbm.at[idx_v.at[ch + 2]], rows_v,
                                     sem)

            return 0

        lax.fori_loop(0, chunks // 2, pair_body, 0)
        pltpu.sync_copy(out_v, out_hbm.at[pl.ds(wid * q_per_w, q_per_w)])

    return kb(table, idx2d)


def _out_body(fr_ref, f0_ref, whs_ref, out_ref):
    diff = fr_ref[0] - f0_ref[0]
    out_ref[0] = lax.dot_general(whs_ref[...], diff, (((1,), (1,)), ((), ())),
                                 preferred_element_type=jnp.float32)


def _out_matmul(f_rows, f0_rows, whs):
    B, N, CP = f_rows.shape
    CH = whs.shape[0]
    return pl.pallas_call(
        _out_body,
        grid=(B, N // TQ),
        in_specs=[
            pl.BlockSpec((1, TQ, CP), lambda b, i: (b, i, 0)),
            pl.BlockSpec((1, TQ, CP), lambda b, i: (b, i, 0)),
            pl.BlockSpec((CH, CP), lambda b, i: (0, 0)),
        ],
        out_specs=pl.BlockSpec((1, CH, TQ), lambda b, i: (b, 0, i)),
        out_shape=jax.ShapeDtypeStruct((B, CH, N), jnp.float32),
    )(f_rows, f0_rows, whs)


def kernel(xyz, feat, W_g, W_h, bn0_w, bn0_b):
    B, N, _ = xyz.shape
    CH = feat.shape[1]
    CP = W_g.shape[0]
    M = W_h.shape[1] // CP

    w_f = W_g[:, :CH]
    w_x = W_g[:, CH:]
    whs = W_h.reshape(CH, M, CP).sum(axis=1)

    wfs, bias = _fold_bn(feat, w_f, bn0_w, bn0_b)
    f0_rows, idx = _feat0_knn(xyz, feat, wfs, w_x, bias)

    table = f0_rows.reshape(B * N, CP)
    idx2d = idx.reshape(B * N * K // 128, 128)
    n_workers = 32
    f_rows = _gather_max_sc(table, idx2d, n_workers, B * N // n_workers, CP)

    return _out_matmul(f_rows.reshape(B, N, CP), f0_rows, whs)
